# baseline (speedup 1.0000x reference)
import functools

import jax
import jax.numpy as jnp
from jax import lax
from jax.experimental import pallas as pl
from jax.experimental.pallas import tpu as pltpu
from jax.experimental.pallas import tpu_sc as plsc

BATCH = 1024
HIDDEN = 64
VOCAB = 100000

_NC = 2
_NS = 16
_NW = _NC * _NS
_B_PER_W = BATCH // _NW


@functools.cache
def _make_sc_gather():
  mesh = plsc.VectorSubcoreMesh(
      core_axis_name="c", subcore_axis_name="s",
      num_cores=_NC, num_subcores=_NS)

  @functools.partial(
      pl.kernel,
      out_type=jax.ShapeDtypeStruct((BATCH, HIDDEN), jnp.float32),
      mesh=mesh,
      scratch_types=[
          pltpu.VMEM((_B_PER_W,), jnp.int32),
          pltpu.VMEM((_B_PER_W, HIDDEN), jnp.float32),
          pltpu.SemaphoreType.DMA,
      ],
      compiler_params=pltpu.CompilerParams(use_tc_tiling_on_sc=False),
  )
  def gather_kernel(idx_hbm, table_hbm, out_hbm, idx_v, rows_v, sem):
    wid = lax.axis_index("s") * _NC + lax.axis_index("c")
    base = wid * _B_PER_W
    pltpu.sync_copy(idx_hbm.at[pl.ds(base, _B_PER_W)], idx_v)
    pltpu.async_copy(table_hbm.at[idx_v], rows_v, sem).wait()
    pltpu.sync_copy(rows_v, out_hbm.at[pl.ds(base, _B_PER_W)])

  return gather_kernel


def _proj_body(x_ref, w_ref, b_ref, out_ref):
  out_ref[...] = jnp.dot(
      x_ref[...], w_ref[...], preferred_element_type=jnp.float32
  ) + b_ref[...]


_BV = 1024


@jax.jit
def kernel(input_ids, emb, W, b):
  x = _make_sc_gather()(input_ids.astype(jnp.int32), emb)

  grid = (pl.cdiv(VOCAB, _BV),)
  logits = pl.pallas_call(
      _proj_body,
      grid=grid,
      in_specs=[
          pl.BlockSpec((BATCH, HIDDEN), lambda j: (0, 0)),
          pl.BlockSpec((HIDDEN, _BV), lambda j: (0, j)),
          pl.BlockSpec((1, _BV), lambda j: (0, j)),
      ],
      out_specs=pl.BlockSpec((BATCH, _BV), lambda j: (0, j)),
      out_shape=jax.ShapeDtypeStruct((BATCH, VOCAB), jnp.float32),
  )(x, W, b.reshape(1, VOCAB))
  return logits


# speedup vs baseline: 1.0312x; 1.0312x over previous
import functools

import jax
import jax.numpy as jnp
from jax import lax
from jax.experimental import pallas as pl
from jax.experimental.pallas import tpu as pltpu
from jax.experimental.pallas import tpu_sc as plsc

BATCH = 1024
HIDDEN = 64
VOCAB = 100000

_NC = 2
_NS = 16
_NW = _NC * _NS
_B_PER_W = BATCH // _NW


@functools.cache
def _make_sc_gather():
  mesh = plsc.VectorSubcoreMesh(
      core_axis_name="c", subcore_axis_name="s",
      num_cores=_NC, num_subcores=_NS)

  @functools.partial(
      pl.kernel,
      out_type=jax.ShapeDtypeStruct((BATCH, HIDDEN), jnp.float32),
      mesh=mesh,
      scratch_types=[
          pltpu.VMEM((_B_PER_W,), jnp.int32),
          pltpu.VMEM((_B_PER_W, HIDDEN), jnp.float32),
          pltpu.SemaphoreType.DMA,
      ],
      compiler_params=pltpu.CompilerParams(use_tc_tiling_on_sc=False),
  )
  def gather_kernel(idx_hbm, table_hbm, out_hbm, idx_v, rows_v, sem):
    wid = lax.axis_index("s") * _NC + lax.axis_index("c")
    base = wid * _B_PER_W
    pltpu.sync_copy(idx_hbm.at[pl.ds(base, _B_PER_W)], idx_v)
    pltpu.async_copy(table_hbm.at[idx_v], rows_v, sem).wait()
    pltpu.sync_copy(rows_v, out_hbm.at[pl.ds(base, _B_PER_W)])

  return gather_kernel


def _proj_body(x_ref, w_ref, b_ref, out_ref):
  out_ref[...] = jnp.dot(
      x_ref[...], w_ref[...], preferred_element_type=jnp.float32
  ) + b_ref[...]


_BB = 32


@jax.jit
def kernel(input_ids, emb, W, b):
  x = _make_sc_gather()(input_ids.astype(jnp.int32), emb)

  grid = (BATCH // _BB,)
  logits = pl.pallas_call(
      _proj_body,
      grid=grid,
      in_specs=[
          pl.BlockSpec((_BB, HIDDEN), lambda i: (i, 0)),
          pl.BlockSpec((HIDDEN, VOCAB), lambda i: (0, 0)),
          pl.BlockSpec((1, VOCAB), lambda i: (0, 0)),
      ],
      out_specs=pl.BlockSpec((_BB, VOCAB), lambda i: (i, 0)),
      out_shape=jax.ShapeDtypeStruct((BATCH, VOCAB), jnp.float32),
  )(x, W, b.reshape(1, VOCAB))
  return logits


# speedup vs baseline: 1.0873x; 1.0544x over previous
import functools

import jax
import jax.numpy as jnp
from jax import lax
from jax.experimental import pallas as pl
from jax.experimental.pallas import tpu as pltpu
from jax.experimental.pallas import tpu_sc as plsc

BATCH = 1024
HIDDEN = 64
VOCAB = 100000

_NC = 2
_NS = 16
_NW = _NC * _NS
_B_PER_W = BATCH // _NW


@functools.cache
def _make_sc_gather():
  mesh = plsc.VectorSubcoreMesh(
      core_axis_name="c", subcore_axis_name="s",
      num_cores=_NC, num_subcores=_NS)

  @functools.partial(
      pl.kernel,
      out_type=jax.ShapeDtypeStruct((BATCH, HIDDEN), jnp.float32),
      mesh=mesh,
      scratch_types=[
          pltpu.VMEM((_B_PER_W,), jnp.int32),
          pltpu.VMEM((_B_PER_W, HIDDEN), jnp.float32),
          pltpu.SemaphoreType.DMA,
      ],
  )
  def gather_kernel(idx_hbm, table_hbm, out_hbm, idx_v, rows_v, sem):
    wid = lax.axis_index("s") * _NC + lax.axis_index("c")
    base = wid * _B_PER_W
    pltpu.sync_copy(idx_hbm.at[pl.ds(base, _B_PER_W)], idx_v)
    copies = []
    for j in range(_B_PER_W):
      vec = idx_v[pl.ds((j // 16) * 16, 16)]
      i = jax.lax.squeeze(
          jax.lax.slice(vec, (j % 16,), (j % 16 + 1,)), (0,))
      c = pltpu.make_async_copy(
          table_hbm.at[pl.ds(i, 1), :], rows_v.at[pl.ds(j, 1), :], sem)
      c.start()
      copies.append(c)
    for c in copies:
      c.wait()
    pltpu.sync_copy(rows_v, out_hbm.at[pl.ds(base, _B_PER_W)])

  return gather_kernel


def _proj_body(x_ref, w_ref, b_ref, out_ref):
  out_ref[...] = jnp.dot(
      x_ref[...], w_ref[...], preferred_element_type=jnp.float32
  ) + b_ref[...]


_BB = 32


@jax.jit
def kernel(input_ids, emb, W, b):
  x = _make_sc_gather()(input_ids.astype(jnp.int32), emb)

  grid = (BATCH // _BB,)
  logits = pl.pallas_call(
      _proj_body,
      grid=grid,
      in_specs=[
          pl.BlockSpec((_BB, HIDDEN), lambda i: (i, 0)),
          pl.BlockSpec((HIDDEN, VOCAB), lambda i: (0, 0)),
          pl.BlockSpec((1, VOCAB), lambda i: (0, 0)),
      ],
      out_specs=pl.BlockSpec((_BB, VOCAB), lambda i: (i, 0)),
      out_shape=jax.ShapeDtypeStruct((BATCH, VOCAB), jnp.float32),
  )(x, W, b.reshape(1, VOCAB))
  return logits


# speedup vs baseline: 2.7578x; 2.5363x over previous
import functools

import jax
import jax.numpy as jnp
from jax import lax
from jax.experimental import pallas as pl
from jax.experimental.pallas import tpu as pltpu
from jax.experimental.pallas import tpu_sc as plsc

BATCH = 1024
HIDDEN = 64
VOCAB = 100000

_NC = 2
_NS = 16
_NW = _NC * _NS
_B_PER_W = BATCH // _NW


@functools.cache
def _make_sc_gather():
  mesh = plsc.VectorSubcoreMesh(
      core_axis_name="c", subcore_axis_name="s",
      num_cores=_NC, num_subcores=_NS)

  @functools.partial(
      pl.kernel,
      out_type=jax.ShapeDtypeStruct((BATCH, HIDDEN), jnp.float32),
      mesh=mesh,
      scratch_types=[
          pltpu.VMEM((_B_PER_W,), jnp.int32),
          pltpu.VMEM((_B_PER_W, HIDDEN), jnp.float32),
          pltpu.SemaphoreType.DMA,
      ],
  )
  def gather_kernel(idx_hbm, table_hbm, out_hbm, idx_v, rows_v, sem):
    wid = lax.axis_index("s") * _NC + lax.axis_index("c")
    base = wid * _B_PER_W
    pltpu.sync_copy(idx_hbm.at[pl.ds(base, _B_PER_W)], idx_v)
    copies = []
    for j in range(_B_PER_W):
      vec = idx_v[pl.ds((j // 16) * 16, 16)]
      i = jax.lax.squeeze(
          jax.lax.slice(vec, (j % 16,), (j % 16 + 1,)), (0,))
      c = pltpu.make_async_copy(
          table_hbm.at[pl.ds(i, 1), :], rows_v.at[pl.ds(j, 1), :], sem)
      c.start()
      copies.append(c)
    for c in copies:
      c.wait()
    pltpu.sync_copy(rows_v, out_hbm.at[pl.ds(base, _B_PER_W)])

  return gather_kernel


_BV = 1024


def _proj_body(x_ref, w_ref, b_ref, out_ref):
  acc = jax.lax.dot_general(
      w_ref[...], x_ref[...],
      dimension_numbers=(((0,), (1,)), ((), ())),
      preferred_element_type=jnp.float32)
  bias = jax.lax.broadcast_in_dim(b_ref[0, :], (_BV, BATCH), (0,))
  out_ref[...] = acc + bias


@jax.jit
def kernel(input_ids, emb, W, b):
  x = _make_sc_gather()(input_ids.astype(jnp.int32), emb)

  logits_t = pl.pallas_call(
      _proj_body,
      grid=(pl.cdiv(VOCAB, _BV),),
      in_specs=[
          pl.BlockSpec((BATCH, HIDDEN), lambda j: (0, 0)),
          pl.BlockSpec((HIDDEN, _BV), lambda j: (0, j)),
          pl.BlockSpec((1, _BV), lambda j: (0, j)),
      ],
      out_specs=pl.BlockSpec((_BV, BATCH), lambda j: (j, 0)),
      out_shape=jax.ShapeDtypeStruct((VOCAB, BATCH), jnp.float32),
  )(x, W, b.reshape(1, VOCAB))
  return logits_t.T


# speedup vs baseline: 3.3414x; 1.2116x over previous
import functools

import jax
import jax.numpy as jnp
from jax import lax
from jax.experimental import pallas as pl
from jax.experimental.pallas import tpu as pltpu
from jax.experimental.pallas import tpu_sc as plsc

BATCH = 1024
HIDDEN = 64
VOCAB = 100000

_NC = 2
_NS = 16
_NW = _NC * _NS
_B_PER_W = BATCH // _NW
_LANES = 128


@functools.cache
def _make_sc_gather():
  mesh = plsc.VectorSubcoreMesh(
      core_axis_name="c", subcore_axis_name="s",
      num_cores=_NC, num_subcores=_NS)

  @functools.partial(
      pl.kernel,
      out_type=jax.ShapeDtypeStruct((BATCH, HIDDEN), jnp.float32),
      mesh=mesh,
      scratch_types=[
          pltpu.VMEM((_B_PER_W,), jnp.int32),
          pltpu.VMEM((HIDDEN, _LANES), jnp.float32),
          pltpu.VMEM((HIDDEN, _LANES), jnp.float32),
          pltpu.VMEM((_B_PER_W, HIDDEN), jnp.float32),
          pltpu.SemaphoreType.DMA,
          pltpu.SemaphoreType.DMA,
      ],
      compiler_params=pltpu.CompilerParams(needs_layout_passes=False),
  )
  def gather_kernel(idx_hbm, table_t_hbm, out_hbm, idx_v, tile0, tile1,
                    rows_v, sem0, sem1):
    wid = lax.axis_index("s") * _NC + lax.axis_index("c")
    base = wid * _B_PER_W
    pltpu.sync_copy(idx_hbm.at[pl.ds(base, _B_PER_W)], idx_v)
    sems = (sem0, sem1)
    tiles = (tile0, tile1)

    def token_id(j):
      vec = idx_v[pl.ds((j // 16) * 16, 16)]
      return jax.lax.squeeze(
          jax.lax.slice(vec, (j % 16,), (j % 16 + 1,)), (0,))

    def fire(j):
      i = token_id(j)
      al = pl.multiple_of((i // _LANES) * _LANES, _LANES)
      c = pltpu.make_async_copy(
          table_t_hbm.at[:, pl.ds(al, _LANES)], tiles[j % 2],
          sems[j % 2])
      c.start()
      return c

    pending = fire(0)
    for j in range(_B_PER_W):
      nxt = fire(j + 1) if j + 1 < _B_PER_W else None
      pending.wait()
      i = token_id(j)
      col = i - (i // _LANES) * _LANES
      colv = jax.lax.broadcast_in_dim(col, (16,), ())
      rowj = jax.lax.broadcast_in_dim(jnp.int32(j), (16,), ())
      for hb in range(HIDDEN // 16):
        rowidx = jax.lax.iota(jnp.int32, 16) + hb * 16
        v = plsc.load_gather(tiles[j % 2], [rowidx, colv])
        plsc.store_scatter(rows_v, [rowj, rowidx], v)
      pending = nxt
    pltpu.sync_copy(rows_v, out_hbm.at[pl.ds(base, _B_PER_W)])

  return gather_kernel


_BV = 5120


def _proj_body(x_ref, w_ref, b_ref, out_ref):
  acc = jax.lax.dot_general(
      w_ref[...], x_ref[...],
      dimension_numbers=(((0,), (1,)), ((), ())),
      preferred_element_type=jnp.float32)
  bias = jax.lax.broadcast_in_dim(b_ref[0, :], (_BV, BATCH), (0,))
  out_ref[...] = acc + bias


@jax.jit
def kernel(input_ids, emb, W, b):
  x = _make_sc_gather()(input_ids.astype(jnp.int32), emb.T)

  logits_t = pl.pallas_call(
      _proj_body,
      grid=(pl.cdiv(VOCAB, _BV),),
      in_specs=[
          pl.BlockSpec((BATCH, HIDDEN), lambda j: (0, 0)),
          pl.BlockSpec((HIDDEN, _BV), lambda j: (0, j)),
          pl.BlockSpec((1, _BV), lambda j: (0, j)),
      ],
      out_specs=pl.BlockSpec((_BV, BATCH), lambda j: (j, 0)),
      out_shape=jax.ShapeDtypeStruct((VOCAB, BATCH), jnp.float32),
  )(x, W, b.reshape(1, VOCAB))
  return logits_t.T


# speedup vs baseline: 3.4487x; 1.0321x over previous
import functools

import jax
import jax.numpy as jnp
from jax import lax
from jax.experimental import pallas as pl
from jax.experimental.pallas import tpu as pltpu
from jax.experimental.pallas import tpu_sc as plsc

BATCH = 1024
HIDDEN = 64
VOCAB = 100000

_NC = 2
_NS = 16
_NW = _NC * _NS
_B_PER_W = BATCH // _NW
_LANES = 128


@functools.cache
def _make_sc_gather():
  mesh = plsc.VectorSubcoreMesh(
      core_axis_name="c", subcore_axis_name="s",
      num_cores=_NC, num_subcores=_NS)

  @functools.partial(
      pl.kernel,
      out_type=jax.ShapeDtypeStruct((BATCH, HIDDEN), jnp.float32),
      mesh=mesh,
      scratch_types=[
          pltpu.VMEM((_B_PER_W,), jnp.int32),
          pltpu.VMEM((HIDDEN, _LANES), jnp.float32),
          pltpu.VMEM((HIDDEN, _LANES), jnp.float32),
          pltpu.VMEM((HIDDEN, _LANES), jnp.float32),
          pltpu.VMEM((HIDDEN, _LANES), jnp.float32),
          pltpu.VMEM((_B_PER_W, HIDDEN), jnp.float32),
          pltpu.SemaphoreType.DMA,
          pltpu.SemaphoreType.DMA,
          pltpu.SemaphoreType.DMA,
          pltpu.SemaphoreType.DMA,
      ],
      compiler_params=pltpu.CompilerParams(needs_layout_passes=False),
  )
  def gather_kernel(idx_hbm, table_t_hbm, out_hbm, idx_v, tile0, tile1,
                    tile2, tile3, rows_v, sem0, sem1, sem2, sem3):
    wid = lax.axis_index("s") * _NC + lax.axis_index("c")
    base = wid * _B_PER_W
    pltpu.sync_copy(idx_hbm.at[pl.ds(base, _B_PER_W)], idx_v)
    sems = (sem0, sem1, sem2, sem3)
    tiles = (tile0, tile1, tile2, tile3)
    _NBUF = 4

    def token_id(j):
      vec = idx_v[pl.ds((j // 16) * 16, 16)]
      return jax.lax.squeeze(
          jax.lax.slice(vec, (j % 16,), (j % 16 + 1,)), (0,))

    def fire(j):
      i = token_id(j)
      al = pl.multiple_of((i // _LANES) * _LANES, _LANES)
      c = pltpu.make_async_copy(
          table_t_hbm.at[:, pl.ds(al, _LANES)], tiles[j % _NBUF],
          sems[j % _NBUF])
      c.start()
      return c

    inflight = [fire(j) for j in range(_NBUF - 1)]
    for j in range(_B_PER_W):
      if j + _NBUF - 1 < _B_PER_W:
        inflight.append(fire(j + _NBUF - 1))
      inflight.pop(0).wait()
      i = token_id(j)
      col = i - (i // _LANES) * _LANES
      colv = jax.lax.broadcast_in_dim(col, (16,), ())
      rowj = jax.lax.broadcast_in_dim(jnp.int32(j), (16,), ())
      for hb in range(HIDDEN // 16):
        rowidx = jax.lax.iota(jnp.int32, 16) + hb * 16
        v = plsc.load_gather(tiles[j % _NBUF], [rowidx, colv])
        plsc.store_scatter(rows_v, [rowj, rowidx], v)
    pltpu.sync_copy(rows_v, out_hbm.at[pl.ds(base, _B_PER_W)])

  return gather_kernel


_BV = 5120


def _proj_body(x_ref, w_ref, b_ref, out_ref):
  acc = jax.lax.dot_general(
      w_ref[...], x_ref[...],
      dimension_numbers=(((0,), (1,)), ((), ())),
      preferred_element_type=jnp.float32)
  bias = jax.lax.broadcast_in_dim(b_ref[0, :], (_BV, BATCH), (0,))
  out_ref[...] = acc + bias


@jax.jit
def kernel(input_ids, emb, W, b):
  x = _make_sc_gather()(input_ids.astype(jnp.int32), emb.T)

  logits_t = pl.pallas_call(
      _proj_body,
      grid=(pl.cdiv(VOCAB, _BV),),
      in_specs=[
          pl.BlockSpec((BATCH, HIDDEN), lambda j: (0, 0)),
          pl.BlockSpec((HIDDEN, _BV), lambda j: (0, j)),
          pl.BlockSpec((1, _BV), lambda j: (0, j)),
      ],
      out_specs=pl.BlockSpec((_BV, BATCH), lambda j: (j, 0)),
      out_shape=jax.ShapeDtypeStruct((VOCAB, BATCH), jnp.float32),
  )(x, W, b.reshape(1, VOCAB))
  return logits_t.T


# speedup vs baseline: 3.4512x; 1.0007x over previous
import functools

import jax
import jax.numpy as jnp
from jax import lax
from jax.experimental import pallas as pl
from jax.experimental.pallas import tpu as pltpu
from jax.experimental.pallas import tpu_sc as plsc

BATCH = 1024
HIDDEN = 64
VOCAB = 100000

_NC = 2
_NS = 16
_NW = _NC * _NS
_B_PER_W = BATCH // _NW
_LANES = 128


@functools.cache
def _make_sc_gather():
  mesh = plsc.VectorSubcoreMesh(
      core_axis_name="c", subcore_axis_name="s",
      num_cores=_NC, num_subcores=_NS)

  @functools.partial(
      pl.kernel,
      out_type=jax.ShapeDtypeStruct((BATCH, HIDDEN), jnp.float32),
      mesh=mesh,
      scratch_types=[
          pltpu.VMEM((_B_PER_W,), jnp.int32),
          pltpu.VMEM((HIDDEN, _LANES), jnp.float32),
          pltpu.VMEM((HIDDEN, _LANES), jnp.float32),
          pltpu.VMEM((HIDDEN, _LANES), jnp.float32),
          pltpu.VMEM((HIDDEN, _LANES), jnp.float32),
          pltpu.VMEM((_B_PER_W, HIDDEN), jnp.float32),
          pltpu.SemaphoreType.DMA,
          pltpu.SemaphoreType.DMA,
          pltpu.SemaphoreType.DMA,
          pltpu.SemaphoreType.DMA,
      ],
      compiler_params=pltpu.CompilerParams(needs_layout_passes=False),
  )
  def gather_kernel(idx_hbm, table_t_hbm, out_hbm, idx_v, tile0, tile1,
                    tile2, tile3, rows_v, sem0, sem1, sem2, sem3):
    wid = lax.axis_index("s") * _NC + lax.axis_index("c")
    base = wid * _B_PER_W
    pltpu.sync_copy(idx_hbm.at[pl.ds(base, _B_PER_W)], idx_v)
    sems = (sem0, sem1, sem2, sem3)
    tiles = (tile0, tile1, tile2, tile3)
    _NBUF = 4

    def token_id(j):
      vec = idx_v[pl.ds((j // 16) * 16, 16)]
      return jax.lax.squeeze(
          jax.lax.slice(vec, (j % 16,), (j % 16 + 1,)), (0,))

    def fire(j):
      i = token_id(j)
      al = pl.multiple_of((i // _LANES) * _LANES, _LANES)
      c = pltpu.make_async_copy(
          table_t_hbm.at[:, pl.ds(al, _LANES)], tiles[j % _NBUF],
          sems[j % _NBUF])
      c.start()
      return c

    inflight = [fire(j) for j in range(_NBUF - 1)]
    for j in range(_B_PER_W):
      if j + _NBUF - 1 < _B_PER_W:
        inflight.append(fire(j + _NBUF - 1))
      inflight.pop(0).wait()
      i = token_id(j)
      col = i - (i // _LANES) * _LANES
      colv = jax.lax.broadcast_in_dim(col, (16,), ())
      rowj = jax.lax.broadcast_in_dim(jnp.int32(j), (16,), ())
      for hb in range(HIDDEN // 16):
        rowidx = jax.lax.iota(jnp.int32, 16) + hb * 16
        v = plsc.load_gather(tiles[j % _NBUF], [rowidx, colv])
        plsc.store_scatter(rows_v, [rowj, rowidx], v)
    pltpu.sync_copy(rows_v, out_hbm.at[pl.ds(base, _B_PER_W)])

  return gather_kernel


_BV = 5632


def _proj_body(x_ref, w_ref, b_ref, out_ref):
  acc = jax.lax.dot_general(
      w_ref[...], x_ref[...],
      dimension_numbers=(((0,), (1,)), ((), ())),
      preferred_element_type=jnp.float32)
  bias = jax.lax.broadcast_in_dim(b_ref[0, :], (_BV, BATCH), (0,))
  out_ref[...] = acc + bias


@jax.jit
def kernel(input_ids, emb, W, b):
  x = _make_sc_gather()(input_ids.astype(jnp.int32), emb.T)

  logits_t = pl.pallas_call(
      _proj_body,
      grid=(pl.cdiv(VOCAB, _BV),),
      in_specs=[
          pl.BlockSpec((BATCH, HIDDEN), lambda j: (0, 0)),
          pl.BlockSpec((HIDDEN, _BV), lambda j: (0, j)),
          pl.BlockSpec((1, _BV), lambda j: (0, j)),
      ],
      out_specs=pl.BlockSpec((_BV, BATCH), lambda j: (j, 0)),
      out_shape=jax.ShapeDtypeStruct((VOCAB, BATCH), jnp.float32),
  )(x, W, b.reshape(1, VOCAB))
  return logits_t.T


# speedup vs baseline: 3.4573x; 1.0017x over previous
import functools

import jax
import jax.numpy as jnp
from jax import lax
from jax.experimental import pallas as pl
from jax.experimental.pallas import tpu as pltpu
from jax.experimental.pallas import tpu_sc as plsc

BATCH = 1024
HIDDEN = 64
VOCAB = 100000

_NC = 2
_NS = 16
_NW = _NC * _NS
_B_PER_W = BATCH // _NW
_LANES = 128


@functools.cache
def _make_sc_gather():
  mesh = plsc.VectorSubcoreMesh(
      core_axis_name="c", subcore_axis_name="s",
      num_cores=_NC, num_subcores=_NS)

  @functools.partial(
      pl.kernel,
      out_type=jax.ShapeDtypeStruct((BATCH, HIDDEN), jnp.float32),
      mesh=mesh,
      scratch_types=[
          pltpu.VMEM((_B_PER_W,), jnp.int32),
          pltpu.VMEM((HIDDEN, _LANES), jnp.float32),
          pltpu.VMEM((HIDDEN, _LANES), jnp.float32),
          pltpu.VMEM((HIDDEN, _LANES), jnp.float32),
          pltpu.VMEM((HIDDEN, _LANES), jnp.float32),
          pltpu.VMEM((_B_PER_W, HIDDEN), jnp.float32),
          pltpu.SemaphoreType.DMA,
          pltpu.SemaphoreType.DMA,
          pltpu.SemaphoreType.DMA,
          pltpu.SemaphoreType.DMA,
      ],
      compiler_params=pltpu.CompilerParams(needs_layout_passes=False),
  )
  def gather_kernel(idx_hbm, table_t_hbm, out_hbm, idx_v, tile0, tile1,
                    tile2, tile3, rows_v, sem0, sem1, sem2, sem3):
    wid = lax.axis_index("s") * _NC + lax.axis_index("c")
    base = wid * _B_PER_W
    pltpu.sync_copy(idx_hbm.at[pl.ds(base, _B_PER_W)], idx_v)
    sems = (sem0, sem1, sem2, sem3)
    tiles = (tile0, tile1, tile2, tile3)
    _NBUF = 4

    def token_id(j):
      vec = idx_v[pl.ds((j // 16) * 16, 16)]
      return jax.lax.squeeze(
          jax.lax.slice(vec, (j % 16,), (j % 16 + 1,)), (0,))

    def fire(j):
      i = token_id(j)
      al = pl.multiple_of((i // _LANES) * _LANES, _LANES)
      c = pltpu.make_async_copy(
          table_t_hbm.at[:, pl.ds(al, _LANES)], tiles[j % _NBUF],
          sems[j % _NBUF])
      c.start()
      return c

    inflight = [fire(j) for j in range(_NBUF - 1)]
    for j in range(_B_PER_W):
      if j + _NBUF - 1 < _B_PER_W:
        inflight.append(fire(j + _NBUF - 1))
      inflight.pop(0).wait()
      i = token_id(j)
      col = i - (i // _LANES) * _LANES
      colv = jax.lax.broadcast_in_dim(col, (16,), ())
      rowj = jax.lax.broadcast_in_dim(jnp.int32(j), (16,), ())
      for hb in range(HIDDEN // 16):
        rowidx = jax.lax.iota(jnp.int32, 16) + hb * 16
        v = plsc.load_gather(tiles[j % _NBUF], [rowidx, colv])
        plsc.store_scatter(rows_v, [rowj, rowidx], v)
    pltpu.sync_copy(rows_v, out_hbm.at[pl.ds(base, _B_PER_W)])

  return gather_kernel


_BV = 5120


def _proj_body(x_ref, w_ref, b_ref, out_ref):
  acc = jax.lax.dot_general(
      w_ref[...], x_ref[...],
      dimension_numbers=(((0,), (1,)), ((), ())),
      preferred_element_type=jnp.float32)
  bias = jax.lax.broadcast_in_dim(b_ref[0, :], (_BV, BATCH), (0,))
  out_ref[...] = acc + bias


@jax.jit
def kernel(input_ids, emb, W, b):
  x = _make_sc_gather()(input_ids.astype(jnp.int32), emb.T)

  logits_t = pl.pallas_call(
      _proj_body,
      grid=(pl.cdiv(VOCAB, _BV),),
      in_specs=[
          pl.BlockSpec((BATCH, HIDDEN), lambda j: (0, 0)),
          pl.BlockSpec((HIDDEN, _BV), lambda j: (0, j)),
          pl.BlockSpec((1, _BV), lambda j: (0, j)),
      ],
      out_specs=pl.BlockSpec((_BV, BATCH), lambda j: (j, 0)),
      out_shape=jax.ShapeDtypeStruct((VOCAB, BATCH), jnp.float32),
  )(x, W, b.reshape(1, VOCAB))
  return logits_t.T
